# baseline (speedup 1.0000x reference)
import functools

import jax
import jax.numpy as jnp
from jax import lax
from jax.experimental import pallas as pl
from jax.experimental.pallas import tpu as pltpu
from jax.experimental.pallas import tpu_sc as plsc

N, C, H, W = 32, 2, 512, 512
HW = H * W
NHW = N * HW
NWORKERS = 32

KSC = 8
NTC = N - KSC

ROWS = 16
RPW = KSC * H // NWORKERS
NCHUNK = RPW // ROWS
VSTEPS = ROWS * W // 16
CPR = W // 16
NBUF = 3

_mesh = plsc.VectorSubcoreMesh(core_axis_name="c", subcore_axis_name="s")


@functools.partial(
    pl.kernel,
    out_type=jax.ShapeDtypeStruct((NWORKERS, 6, 16), jnp.float32),
    mesh=_mesh,
    scratch_types=(
        [pltpu.VMEM((ROWS, W), jnp.float32) for _ in range(NBUF)]
        + [pltpu.VMEM((ROWS, W), jnp.float32) for _ in range(NBUF)]
        + [pltpu.VMEM((ROWS, W), jnp.int32) for _ in range(NBUF)]
        + [pltpu.VMEM((6, 16), jnp.float32)]
        + [pltpu.SemaphoreType.DMA for _ in range(NBUF)]
    ),
)
def _dice_partials_sc(in_hbm, t_hbm, out_hbm, *refs):
    a0bufs = refs[0:NBUF]
    a1bufs = refs[NBUF:2 * NBUF]
    tbufs = refs[2 * NBUF:3 * NBUF]
    vout = refs[3 * NBUF]
    sems = refs[3 * NBUF + 1:]

    wid = lax.axis_index("s") * 2 + lax.axis_index("c")
    batch = wid * RPW // H
    row0 = (wid * RPW) % H

    def issue(g, b):
        r = pl.ds(row0 + g * ROWS, ROWS)
        return (
            pltpu.async_copy(in_hbm.at[batch, 0, r, :], a0bufs[b], sems[b]),
            pltpu.async_copy(in_hbm.at[batch, 1, r, :], a1bufs[b], sems[b]),
            pltpu.async_copy(t_hbm.at[batch, r, :], tbufs[b], sems[b]),
        )

    pend = [issue(g, g) for g in range(min(NBUF, NCHUNK))]

    zf = jnp.zeros((16,), jnp.float32)
    zi = jnp.zeros((16,), jnp.int32)
    acc = (zf, zf, zf, zf, zi)

    for g in range(NCHUNK):
        b = g % NBUF
        for hnd in pend[b]:
            hnd.wait()
        a0r, a1r, tr = a0bufs[b], a1bufs[b], tbufs[b]

        def body(i, carry, a0r=a0r, a1r=a1r, tr=tr):
            q, r_, sq0, sq1, cnt = carry
            row = lax.shift_right_logical(i, 5)
            col = pl.multiple_of(lax.shift_left(lax.bitwise_and(i, CPR - 1), 4), 16)
            sl = pl.ds(col, 16)
            a0 = a0r[row, sl]
            a1 = a1r[row, sl]
            tv = tr[row, sl]
            m = tv != 0
            return (q + jnp.where(m, a1, a0), r_ + jnp.where(m, a1, zf),
                    sq0 + a0 * a0, sq1 + a1 * a1, cnt + tv)

        acc = lax.fori_loop(0, VSTEPS, body, acc, unroll=4)
        if g + NBUF < NCHUNK:
            pend[b] = issue(g + NBUF, b)

    q, r_, sq0, sq1, cnt = acc
    for row, v in enumerate((q, r_, sq0, sq1, cnt.astype(jnp.float32), zf)):
        vout[row, :] = v
    pltpu.sync_copy(vout, out_hbm.at[wid])


def _dice_tc_body(in_ref, t_ref, out_ref):
    @pl.when(pl.program_id(0) == 0)
    def _init():
        out_ref[...] = jnp.zeros_like(out_ref)

    a0 = in_ref[0, 0]
    a1 = in_ref[0, 1]
    tv = t_ref[0]
    m = tv != 0
    q = jnp.sum(jnp.where(m, a1, a0))
    r = jnp.sum(jnp.where(m, a1, 0.0))
    sq0 = jnp.sum(a0 * a0)
    sq1 = jnp.sum(a1 * a1)
    cnt = jnp.sum(tv).astype(jnp.float32)
    tiles = jnp.stack([jnp.full((8, 128), v, jnp.float32)
                       for v in (q, r, sq0, sq1, cnt)])
    out_ref[...] += tiles


_dice_partials_tc = pl.pallas_call(
    _dice_tc_body,
    grid=(NTC,),
    in_specs=[
        pl.BlockSpec((1, C, H, W), lambda i: (i + KSC, 0, 0, 0)),
        pl.BlockSpec((1, H, W), lambda i: (i + KSC, 0, 0)),
    ],
    out_specs=pl.BlockSpec((5, 8, 128), lambda i: (0, 0, 0)),
    out_shape=jax.ShapeDtypeStruct((5, 8, 128), jnp.float32),
)


def kernel(inputs, targets, smooth):
    t32 = targets.astype(jnp.int32)
    parts_sc = _dice_partials_sc(inputs, t32)
    parts_tc = _dice_partials_tc(inputs, t32)
    q_s, r_s, sq0_s, sq1_s, cnt_s, _ = jnp.sum(parts_sc, axis=(0, 2))
    q = q_s + parts_tc[0, 0, 0]
    r = r_s + parts_tc[1, 0, 0]
    sq0 = sq0_s + parts_tc[2, 0, 0]
    sq1 = sq1_s + parts_tc[3, 0, 0]
    cnt = cnt_s + parts_tc[4, 0, 0]
    sm = smooth.astype(jnp.float32)
    loss0 = 1.0 - (2.0 * (q - r) + sm) / (sq0 + (NHW - cnt) + sm)
    loss1 = 1.0 - (2.0 * r + sm) / (sq1 + cnt + sm)
    return (loss0 + loss1) * 0.5


# speedup vs baseline: 1.2633x; 1.2633x over previous
import functools

import jax
import jax.numpy as jnp
from jax import lax
from jax.experimental import pallas as pl
from jax.experimental.pallas import tpu as pltpu
from jax.experimental.pallas import tpu_sc as plsc

N, C, H, W = 32, 2, 512, 512
HW = H * W
NHW = N * HW
NWORKERS = 32

KSC = 0
NTC = N - KSC

ROWS = 16
RPW = KSC * H // NWORKERS
NCHUNK = RPW // ROWS
VSTEPS = ROWS * W // 16
CPR = W // 16
NBUF = 3

_mesh = plsc.VectorSubcoreMesh(core_axis_name="c", subcore_axis_name="s")


@functools.partial(
    pl.kernel,
    out_type=jax.ShapeDtypeStruct((NWORKERS, 6, 16), jnp.float32),
    mesh=_mesh,
    scratch_types=(
        [pltpu.VMEM((ROWS, W), jnp.float32) for _ in range(NBUF)]
        + [pltpu.VMEM((ROWS, W), jnp.float32) for _ in range(NBUF)]
        + [pltpu.VMEM((ROWS, W), jnp.int32) for _ in range(NBUF)]
        + [pltpu.VMEM((6, 16), jnp.float32)]
        + [pltpu.SemaphoreType.DMA for _ in range(NBUF)]
    ),
)
def _dice_partials_sc(in_hbm, t_hbm, out_hbm, *refs):
    a0bufs = refs[0:NBUF]
    a1bufs = refs[NBUF:2 * NBUF]
    tbufs = refs[2 * NBUF:3 * NBUF]
    vout = refs[3 * NBUF]
    sems = refs[3 * NBUF + 1:]

    wid = lax.axis_index("s") * 2 + lax.axis_index("c")
    batch = wid * RPW // H
    row0 = (wid * RPW) % H

    def issue(g, b):
        r = pl.ds(row0 + g * ROWS, ROWS)
        return (
            pltpu.async_copy(in_hbm.at[batch, 0, r, :], a0bufs[b], sems[b]),
            pltpu.async_copy(in_hbm.at[batch, 1, r, :], a1bufs[b], sems[b]),
            pltpu.async_copy(t_hbm.at[batch, r, :], tbufs[b], sems[b]),
        )

    pend = [issue(g, g) for g in range(min(NBUF, NCHUNK))]

    zf = jnp.zeros((16,), jnp.float32)
    zi = jnp.zeros((16,), jnp.int32)
    acc = (zf, zf, zf, zf, zi)

    for g in range(NCHUNK):
        b = g % NBUF
        for hnd in pend[b]:
            hnd.wait()
        a0r, a1r, tr = a0bufs[b], a1bufs[b], tbufs[b]

        def body(i, carry, a0r=a0r, a1r=a1r, tr=tr):
            q, r_, sq0, sq1, cnt = carry
            row = lax.shift_right_logical(i, 5)
            col = pl.multiple_of(lax.shift_left(lax.bitwise_and(i, CPR - 1), 4), 16)
            sl = pl.ds(col, 16)
            a0 = a0r[row, sl]
            a1 = a1r[row, sl]
            tv = tr[row, sl]
            m = tv != 0
            return (q + jnp.where(m, a1, a0), r_ + jnp.where(m, a1, zf),
                    sq0 + a0 * a0, sq1 + a1 * a1, cnt + tv)

        acc = lax.fori_loop(0, VSTEPS, body, acc, unroll=4)
        if g + NBUF < NCHUNK:
            pend[b] = issue(g + NBUF, b)

    q, r_, sq0, sq1, cnt = acc
    for row, v in enumerate((q, r_, sq0, sq1, cnt.astype(jnp.float32), zf)):
        vout[row, :] = v
    pltpu.sync_copy(vout, out_hbm.at[wid])


def _dice_tc_body(in_ref, t_ref, out_ref):
    @pl.when(pl.program_id(0) == 0)
    def _init():
        out_ref[...] = jnp.zeros_like(out_ref)

    a0 = in_ref[0, 0]
    a1 = in_ref[0, 1]
    tv = t_ref[0]
    m = tv != 0
    q = jnp.sum(jnp.where(m, a1, a0))
    r = jnp.sum(jnp.where(m, a1, 0.0))
    sq0 = jnp.sum(a0 * a0)
    sq1 = jnp.sum(a1 * a1)
    cnt = jnp.sum(tv).astype(jnp.float32)
    tiles = jnp.stack([jnp.full((8, 128), v, jnp.float32)
                       for v in (q, r, sq0, sq1, cnt)])
    out_ref[...] += tiles


_dice_partials_tc = pl.pallas_call(
    _dice_tc_body,
    grid=(NTC,),
    in_specs=[
        pl.BlockSpec((1, C, H, W), lambda i: (i + KSC, 0, 0, 0)),
        pl.BlockSpec((1, H, W), lambda i: (i + KSC, 0, 0)),
    ],
    out_specs=pl.BlockSpec((5, 8, 128), lambda i: (0, 0, 0)),
    out_shape=jax.ShapeDtypeStruct((5, 8, 128), jnp.float32),
)


def kernel(inputs, targets, smooth):
    t32 = targets.astype(jnp.int32)
    parts_tc = _dice_partials_tc(inputs, t32)
    q_s = r_s = sq0_s = sq1_s = cnt_s = jnp.float32(0.0)
    q = q_s + parts_tc[0, 0, 0]
    r = r_s + parts_tc[1, 0, 0]
    sq0 = sq0_s + parts_tc[2, 0, 0]
    sq1 = sq1_s + parts_tc[3, 0, 0]
    cnt = cnt_s + parts_tc[4, 0, 0]
    sm = smooth.astype(jnp.float32)
    loss0 = 1.0 - (2.0 * (q - r) + sm) / (sq0 + (NHW - cnt) + sm)
    loss1 = 1.0 - (2.0 * r + sm) / (sq1 + cnt + sm)
    return (loss0 + loss1) * 0.5
